# baseline (speedup 1.0000x reference)
import functools

import jax
import jax.numpy as jnp
from jax import lax
from jax.experimental import pallas as pl
from jax.experimental.pallas import tpu as pltpu
from jax.experimental.pallas import tpu_sc as plsc

_NEG_MARGIN = 3.0
_TB = 8
_LANES = 16


def kernel(a1_embedding, a2_embedding, a1_align, a2_align,
           neg1_left, neg1_right, neg2_left, neg2_right):
    T = a1_align.shape[0]
    N = neg1_left.shape[0]
    D = a1_embedding.shape[1]
    K = N // T
    assert N == T * K and T % _TB == 0 and D % _LANES == 0
    NB = _TB * K
    CB = _TB + NB
    DV = D // _LANES

    info = plsc.get_sparse_core_info()
    NC, NS = info.num_cores, info.num_subcores
    NW = NC * NS
    n_chunks = T // _TB
    NK = -(-n_chunks // NW)
    if NK % 2:
        NK += 1
    CSL = NK * CB
    NSL = NK * NB

    mesh = plsc.VectorSubcoreMesh(core_axis_name="c", subcore_axis_name="s")

    @functools.partial(
        pl.kernel,
        out_type=jax.ShapeDtypeStruct((NW, _LANES), jnp.float32),
        mesh=mesh,
        compiler_params=pltpu.CompilerParams(needs_layout_passes=False),
        scratch_types=[
            pltpu.VMEM((CSL,), jnp.int32),
            pltpu.VMEM((CSL,), jnp.int32),
            pltpu.VMEM((NSL,), jnp.int32),
            pltpu.VMEM((NSL,), jnp.int32),
            pltpu.VMEM((2, CB, D), jnp.float32),
            pltpu.VMEM((2, CB, D), jnp.float32),
            pltpu.VMEM((2, NB, D), jnp.float32),
            pltpu.VMEM((2, NB, D), jnp.float32),
            pltpu.VMEM((_LANES,), jnp.float32),
            pltpu.SemaphoreType.DMA,
            pltpu.SemaphoreType.DMA,
        ],
    )
    def sc_loss(a1e, a2e, c1h, c2h, n2lh, n2rh, out_hbm,
                c1v, c2v, j2l, j2r,
                g1, g2, r2l, r2r, ostage, sem0, sem1):
        w = lax.axis_index("s") * NC + lax.axis_index("c")
        cb = jnp.minimum(w * NK, n_chunks - NK)

        pltpu.sync_copy(c1h.at[pl.ds(cb * CB, CSL)], c1v)
        pltpu.sync_copy(c2h.at[pl.ds(cb * CB, CSL)], c2v)
        pltpu.sync_copy(n2lh.at[pl.ds(cb * NB, NSL)], j2l)
        pltpu.sync_copy(n2rh.at[pl.ds(cb * NB, NSL)], j2r)

        sems = (sem0, sem1)

        def descriptors(d, k):
            c = jnp.minimum(w * NK + k, n_chunks - 1)
            co = (c - cb) * CB
            no = (c - cb) * NB
            sem = sems[d]
            return [
                (a1e.at[c1v.at[pl.ds(co, CB)]], g1.at[d], sem),
                (a2e.at[c2v.at[pl.ds(co, CB)]], g2.at[d], sem),
                (a1e.at[j2l.at[pl.ds(no, NB)]], r2l.at[d], sem),
                (a2e.at[j2r.at[pl.ds(no, NB)]], r2r.at[d], sem),
            ]

        def issue(d, k):
            for src, dst, sem in descriptors(d, k):
                pltpu.async_copy(src, dst, sem)

        def drain(d, k):
            for src, dst, sem in descriptors(d, k):
                pltpu.make_async_copy(src, dst, sem).wait()

        def l1_terms(ref_l, ref_r, d, i):
            acc = [None, None]
            for v in range(DV):
                sl = pl.ds(v * _LANES, _LANES)
                term = jnp.abs(ref_l[d, i, sl] - ref_r[d, i, sl])
                acc[v % 2] = term if acc[v % 2] is None else acc[v % 2] + term
            return acc[0], acc[1]

        def compute(d):
            def t_body(t, csum):
                pe, po = l1_terms(g1, g2, d, t)
                pos_s = jnp.sum(pe + po)
                for j in range(K):
                    i = t * K + j
                    e1, o1 = l1_terms(g1, g2, d, _TB + i)
                    e2, o2 = l1_terms(r2l, r2r, d, i)
                    arg = pos_s + _NEG_MARGIN - jnp.sum((e1 + o1) + (e2 + o2))
                    csum = csum + jnp.maximum(arg, jnp.float32(0.0))
                return csum
            return lax.fori_loop(0, _TB, t_body, jnp.float32(0.0))

        def valid(k):
            return w * NK + k < n_chunks

        issue(0, 0)

        def pipe_body(k2, total):
            k = 2 * k2
            issue(1, k + 1)
            drain(0, k)
            total = total + jnp.where(valid(k), compute(0), jnp.float32(0.0))

            @pl.when(k + 2 < NK)
            def _():
                issue(0, k + 2)

            drain(1, k + 1)
            total = total + jnp.where(valid(k + 1), compute(1), jnp.float32(0.0))
            return total

        total = lax.fori_loop(0, NK // 2, pipe_body, jnp.float32(0.0))

        lanes = lax.iota(jnp.int32, _LANES)
        ostage[...] = jnp.where(lanes == 0, total, jnp.float32(0.0))
        pltpu.sync_copy(ostage, out_hbm.at[w])

    def comb(pos_idx, neg_idx):
        return jnp.concatenate(
            [pos_idx.astype(jnp.int32).reshape(n_chunks, _TB),
             neg_idx.astype(jnp.int32).reshape(n_chunks, NB)], axis=1
        ).reshape(-1)

    partials = sc_loss(a1_embedding, a2_embedding,
                       comb(a1_align, neg1_left), comb(a2_align, neg1_right),
                       neg2_left.astype(jnp.int32), neg2_right.astype(jnp.int32))
    return jnp.sum(partials)


# speedup vs baseline: 1.0237x; 1.0237x over previous
import functools

import jax
import jax.numpy as jnp
from jax import lax
from jax.experimental import pallas as pl
from jax.experimental.pallas import tpu as pltpu
from jax.experimental.pallas import tpu_sc as plsc

_NEG_MARGIN = 3.0
_TB = 8
_LANES = 16


def kernel(a1_embedding, a2_embedding, a1_align, a2_align,
           neg1_left, neg1_right, neg2_left, neg2_right):
    T = a1_align.shape[0]
    N = neg1_left.shape[0]
    D = a1_embedding.shape[1]
    K = N // T
    assert N == T * K and T % _TB == 0 and D % _LANES == 0
    NB = _TB * K
    DV = D // _LANES

    info = plsc.get_sparse_core_info()
    NC, NS = info.num_cores, info.num_subcores
    NW = NC * NS
    n_chunks = T // _TB
    NK = -(-n_chunks // NW)
    if NK % 2:
        NK += 1
    TS = NK * _TB
    NSL = NK * NB

    mesh = plsc.VectorSubcoreMesh(core_axis_name="c", subcore_axis_name="s")

    @functools.partial(
        pl.kernel,
        out_type=jax.ShapeDtypeStruct((NW, _LANES), jnp.float32),
        mesh=mesh,
        compiler_params=pltpu.CompilerParams(needs_layout_passes=False),
        scratch_types=[
            pltpu.VMEM((TS,), jnp.int32),
            pltpu.VMEM((TS,), jnp.int32),
            pltpu.VMEM((NSL,), jnp.int32),
            pltpu.VMEM((NSL,), jnp.int32),
            pltpu.VMEM((NSL,), jnp.int32),
            pltpu.VMEM((NSL,), jnp.int32),
            pltpu.VMEM((2, _TB, D), jnp.float32),
            pltpu.VMEM((2, _TB, D), jnp.float32),
            pltpu.VMEM((2, NB, D), jnp.float32),
            pltpu.VMEM((2, NB, D), jnp.float32),
            pltpu.VMEM((2, NB, D), jnp.float32),
            pltpu.VMEM((2, NB, D), jnp.float32),
            pltpu.VMEM((_LANES,), jnp.float32),
            pltpu.SemaphoreType.DMA,
            pltpu.SemaphoreType.DMA,
        ],
    )
    def sc_loss(a1e, a2e, i1h, i2h, n1lh, n1rh, n2lh, n2rh, out_hbm,
                i1v, i2v, j1l, j1r, j2l, j2r,
                p1, p2, r1l, r1r, r2l, r2r, ostage, sem0, sem1):
        w = lax.axis_index("s") * NC + lax.axis_index("c")
        t_base = jnp.minimum(w * TS, T - TS)
        n_base = t_base * K

        pltpu.sync_copy(i1h.at[pl.ds(t_base, TS)], i1v)
        pltpu.sync_copy(i2h.at[pl.ds(t_base, TS)], i2v)
        pltpu.sync_copy(n1lh.at[pl.ds(n_base, NSL)], j1l)
        pltpu.sync_copy(n1rh.at[pl.ds(n_base, NSL)], j1r)
        pltpu.sync_copy(n2lh.at[pl.ds(n_base, NSL)], j2l)
        pltpu.sync_copy(n2rh.at[pl.ds(n_base, NSL)], j2r)

        sems = (sem0, sem1)

        def chunk_offsets(k):
            c = jnp.minimum(w * NK + k, n_chunks - 1)
            return c * _TB - t_base, c * NB - n_base

        def descriptors(d, k):
            lt, ln = chunk_offsets(k)
            sem = sems[d]
            return [
                (a1e.at[i1v.at[pl.ds(lt, _TB)]], p1.at[d], sem),
                (a2e.at[i2v.at[pl.ds(lt, _TB)]], p2.at[d], sem),
                (a1e.at[j1l.at[pl.ds(ln, NB)]], r1l.at[d], sem),
                (a2e.at[j1r.at[pl.ds(ln, NB)]], r1r.at[d], sem),
                (a1e.at[j2l.at[pl.ds(ln, NB)]], r2l.at[d], sem),
                (a2e.at[j2r.at[pl.ds(ln, NB)]], r2r.at[d], sem),
            ]

        def issue(d, k):
            for src, dst, sem in descriptors(d, k):
                pltpu.async_copy(src, dst, sem)

        def drain(d, k):
            for src, dst, sem in descriptors(d, k):
                pltpu.make_async_copy(src, dst, sem).wait()

        def l1_terms(ref_l, ref_r, d, i):
            acc = [None, None]
            for v in range(DV):
                sl = pl.ds(v * _LANES, _LANES)
                term = jnp.abs(ref_l[d, i, sl] - ref_r[d, i, sl])
                acc[v % 2] = term if acc[v % 2] is None else acc[v % 2] + term
            return acc[0], acc[1]

        def compute(d):
            def t_body(t, csum):
                pe, po = l1_terms(p1, p2, d, t)
                pos_s = jnp.sum(pe + po)
                for j in range(K):
                    i = t * K + j
                    e1, o1 = l1_terms(r1l, r1r, d, i)
                    e2, o2 = l1_terms(r2l, r2r, d, i)
                    arg = pos_s + _NEG_MARGIN - jnp.sum((e1 + o1) + (e2 + o2))
                    csum = csum + jnp.maximum(arg, jnp.float32(0.0))
                return csum
            return lax.fori_loop(0, _TB, t_body, jnp.float32(0.0))

        def valid(k):
            return w * NK + k < n_chunks

        issue(0, 0)

        def pipe_body(k2, total):
            k = 2 * k2
            issue(1, k + 1)
            drain(0, k)
            total = total + jnp.where(valid(k), compute(0), jnp.float32(0.0))

            @pl.when(k + 2 < NK)
            def _():
                issue(0, k + 2)

            drain(1, k + 1)
            total = total + jnp.where(valid(k + 1), compute(1), jnp.float32(0.0))
            return total

        total = lax.fori_loop(0, NK // 2, pipe_body, jnp.float32(0.0))

        lanes = lax.iota(jnp.int32, _LANES)
        ostage[...] = jnp.where(lanes == 0, total, jnp.float32(0.0))
        pltpu.sync_copy(ostage, out_hbm.at[w])

    partials = sc_loss(a1_embedding, a2_embedding,
                       a1_align.astype(jnp.int32), a2_align.astype(jnp.int32),
                       neg1_left.astype(jnp.int32), neg1_right.astype(jnp.int32),
                       neg2_left.astype(jnp.int32), neg2_right.astype(jnp.int32))
    return jnp.sum(partials)
